# speedup vs baseline: 1.3633x; 1.3633x over previous
import functools

import jax
import jax.numpy as jnp
from jax.experimental import pallas as pl

_MIN32 = -2147483648


def _matvec_kernel(x_ref, w_ref, o_ref):
    o_ref[...] = jnp.dot(x_ref[...], w_ref[...],
                         preferred_element_type=jnp.float32)


def _mask_kernel(s_ref, o_ref, *, k, t):
    s = s_ref[...]
    rows = s.shape[0]
    u = jax.lax.bitcast_convert_type(s, jnp.int32)
    skey = u ^ (jax.lax.shift_right_arithmetic(u, 31) & jnp.int32(0x7FFFFFFF))
    min32 = jnp.int32(_MIN32)

    def vbody(i, p):
        cand = p | jnp.left_shift(jnp.int32(1), 31 - i)
        thr = cand ^ min32
        cnt = jnp.sum((skey >= thr).astype(jnp.int32), axis=1, keepdims=True)
        return jnp.where(cnt >= k, cand, p)

    p = jax.lax.fori_loop(0, 32, vbody, jnp.zeros((rows, 1), jnp.int32))
    kth = p ^ min32

    gt = skey > kth
    eq = skey == kth
    need = k - jnp.sum(gt.astype(jnp.int32), axis=1, keepdims=True)
    idx = jax.lax.broadcasted_iota(jnp.int32, s.shape, 1)

    def ibody(i, tv):
        cand = tv + jnp.left_shift(jnp.int32(1), 12 - i)
        cnt = jnp.sum((eq & (idx < cand)).astype(jnp.int32),
                      axis=1, keepdims=True)
        ok = (cand <= t) & (cnt <= need)
        return jnp.where(ok, cand, tv)

    tv = jax.lax.fori_loop(0, 13, ibody, jnp.zeros((rows, 1), jnp.int32))
    mask = gt | (eq & (idx < tv))
    o_ref[...] = mask.astype(jnp.float32)


def kernel(x, W):
    b, t, d = x.shape
    k = max(1, int(t * 0.5))
    x2 = x.reshape(b * t, d)
    tile = 1024
    grid = (b * t) // tile

    scores_col = pl.pallas_call(
        _matvec_kernel,
        grid=(grid,),
        in_specs=[
            pl.BlockSpec((tile, d), lambda i: (i, 0)),
            pl.BlockSpec((d, 1), lambda i: (0, 0)),
        ],
        out_specs=pl.BlockSpec((tile, 1), lambda i: (i, 0)),
        out_shape=jax.ShapeDtypeStruct((b * t, 1), jnp.float32),
    )(x2, W)
    scores = scores_col.reshape(b, t)

    mask2 = pl.pallas_call(
        functools.partial(_mask_kernel, k=k, t=t),
        out_shape=jax.ShapeDtypeStruct((b, t), jnp.float32),
    )(scores)
    return (mask2.reshape(b, t, 1), scores)
